# baseline (speedup 1.0000x reference)
import functools

import jax
import jax.numpy as jnp
from jax import lax
from jax.experimental import pallas as pl
from jax.experimental.pallas import tpu as pltpu
from jax.experimental.pallas import tpu_sc as plsc

D = 128
HID = 128
EB = 256
KC = 16
NC = 2
NS = 16
NW = NC * NS
EPW = 640
ROWCH = 128

_HIGH = lax.Precision.HIGHEST


def _gather_body(x_hbm, src_hbm, out_hbm, idx_v, rows_v, sem):
    c = lax.axis_index("c")
    s = lax.axis_index("s")
    wid = c * NS + s
    pltpu.sync_copy(src_hbm.at[wid], idx_v)
    for j in range(EPW // ROWCH):
        pltpu.async_copy(x_hbm.at[idx_v.at[j]],
                         rows_v.at[pl.ds(j * ROWCH, ROWCH)], sem).wait()
    pltpu.sync_copy(rows_v, out_hbm.at[pl.ds(wid * EPW, EPW)])


def _sc_gather(x, src_r, e_pad):
    mesh = plsc.VectorSubcoreMesh(core_axis_name="c", subcore_axis_name="s")
    k = pl.kernel(
        _gather_body,
        out_type=jax.ShapeDtypeStruct((e_pad, D), jnp.float32),
        mesh=mesh,
        scratch_types=[
            pltpu.VMEM((EPW // ROWCH, ROWCH), jnp.int32),
            pltpu.VMEM((EPW, D), jnp.float32),
            pltpu.SemaphoreType.DMA,
        ],
    )
    return k(x, src_r)


HALF = 5120
ACC1 = 5248
ACC2 = 128


def _scatter_body(msg_hbm, hot_hbm, idxr_hbm, idxhi_hbm, zero_hbm,
                  out1_hbm, out2_hbm, idx_v, hi_v, rows_v, acc_sh, cnt_sh):
    per1 = ACC1 // NS
    per2 = ACC2 // NS
    c = lax.axis_index("c")
    s = lax.axis_index("s")

    pltpu.sync_copy(zero_hbm, acc_sh.at[pl.ds(s * per1, per1)])
    pltpu.sync_copy(zero_hbm.at[pl.ds(0, per2)],
                    cnt_sh.at[pl.ds(s * per2, per2)])
    pltpu.sync_copy(idxr_hbm.at[c, s], idx_v)
    pltpu.sync_copy(idxhi_hbm.at[c, s], hi_v)
    plsc.subcore_barrier()

    for chunk in range(2):
        base = s * 2 * EPW + chunk * EPW
        pltpu.sync_copy(msg_hbm.at[pl.ds(base, EPW)], rows_v)
        for j in range(EPW // ROWCH):
            pltpu.sync_copy(rows_v.at[pl.ds(j * ROWCH, ROWCH)],
                            acc_sh.at[idx_v.at[chunk * (EPW // ROWCH) + j]],
                            add=True)
        pltpu.sync_copy(hot_hbm.at[pl.ds(base, EPW)], rows_v)
        for j in range(EPW // ROWCH):
            pltpu.sync_copy(rows_v.at[pl.ds(j * ROWCH, ROWCH)],
                            cnt_sh.at[hi_v.at[chunk * (EPW // ROWCH) + j]],
                            add=True)

    plsc.subcore_barrier()
    pltpu.sync_copy(acc_sh.at[pl.ds(s * per1, per1)],
                    out1_hbm.at[c, pl.ds(s * per1, per1)])
    pltpu.sync_copy(cnt_sh.at[pl.ds(s * per2, per2)],
                    out2_hbm.at[c, pl.ds(s * per2, per2)])


def _sc_scatter(msg, hot, idxr, idxhi, zeros):
    mesh = plsc.VectorSubcoreMesh(core_axis_name="c", subcore_axis_name="s")
    k = pl.kernel(
        _scatter_body,
        out_type=(jax.ShapeDtypeStruct((NC, ACC1, D), jnp.float32),
                  jax.ShapeDtypeStruct((NC, ACC2, D), jnp.float32)),
        mesh=mesh,
        scratch_types=[
            pltpu.VMEM((2 * EPW // ROWCH, ROWCH), jnp.int32),
            pltpu.VMEM((2 * EPW // ROWCH, ROWCH), jnp.int32),
            pltpu.VMEM((EPW, D), jnp.float32),
            pltpu.VMEM_SHARED((ACC1, D), jnp.float32),
            pltpu.VMEM_SHARED((ACC2, D), jnp.float32),
        ],
    )
    return k(msg, hot, idxr, idxhi, zeros)


def _edge_kernel(ea_ref, xj_ref, dm_ref, W1_ref, b1_ref, M_ref, B2_ref,
                 msg_ref, hot_ref, gc_ref):
    h = jnp.maximum(
        jnp.dot(ea_ref[...], W1_ref[...], preferred_element_type=jnp.float32,
                precision=_HIGH) + b1_ref[...], 0.0)
    xj = xj_ref[...]
    acc = jnp.dot(xj, B2_ref[...], preferred_element_type=jnp.float32,
                  precision=_HIGH)
    for kc in range(0, HID, KC):
        for j in range(KC):
            gc_ref[:, j * D:(j + 1) * D] = h[:, kc + j:kc + j + 1] * xj
        acc = acc + jnp.dot(
            gc_ref[...], M_ref[kc * D:(kc + KC) * D, :],
            preferred_element_type=jnp.float32, precision=_HIGH)
    msg_ref[...] = acc
    lanes = lax.broadcasted_iota(jnp.int32, (EB, D), 1)
    hot_ref[...] = jnp.where(lanes == dm_ref[...], 1.0, 0.0)


def _tc_edge(ea, xj, dmod, W1, b1, M, B2, e_pad):
    grid = e_pad // EB
    return pl.pallas_call(
        _edge_kernel,
        grid=(grid,),
        in_specs=[
            pl.BlockSpec((EB, D), lambda i: (i, 0)),
            pl.BlockSpec((EB, D), lambda i: (i, 0)),
            pl.BlockSpec((EB, 1), lambda i: (i, 0)),
            pl.BlockSpec((D, HID), lambda i: (0, 0)),
            pl.BlockSpec((1, HID), lambda i: (0, 0)),
            pl.BlockSpec((HID * D, D), lambda i: (0, 0)),
            pl.BlockSpec((D, D), lambda i: (0, 0)),
        ],
        out_specs=[
            pl.BlockSpec((EB, D), lambda i: (i, 0)),
            pl.BlockSpec((EB, D), lambda i: (i, 0)),
        ],
        out_shape=[
            jax.ShapeDtypeStruct((e_pad, D), jnp.float32),
            jax.ShapeDtypeStruct((e_pad, D), jnp.float32),
        ],
        scratch_shapes=[pltpu.VMEM((EB, KC * D), jnp.float32)],
    )(ea, xj, dmod, W1, b1, M, B2)


def _final_kernel(x_ref, s_ref, cnt_ref, root_ref, bias_ref, out_ref):
    x = x_ref[...]
    aggr = s_ref[...] / jnp.maximum(cnt_ref[...], 1.0)
    pre = aggr + jnp.dot(x, root_ref[...], preferred_element_type=jnp.float32,
                         precision=_HIGH) + bias_ref[...]
    out_ref[...] = x + 0.5 * pre * (1.0 + lax.erf(pre * 0.7071067811865476))


def _tc_final(x, summed, cnt, root, bias, n):
    nb = 1000
    return pl.pallas_call(
        _final_kernel,
        grid=(n // nb,),
        in_specs=[
            pl.BlockSpec((nb, D), lambda i: (i, 0)),
            pl.BlockSpec((nb, D), lambda i: (i, 0)),
            pl.BlockSpec((nb, 1), lambda i: (i, 0)),
            pl.BlockSpec((D, D), lambda i: (0, 0)),
            pl.BlockSpec((1, D), lambda i: (0, 0)),
        ],
        out_specs=pl.BlockSpec((nb, D), lambda i: (i, 0)),
        out_shape=jax.ShapeDtypeStruct((n, D), jnp.float32),
    )(x, summed, cnt, root, bias)


def kernel(x, edge_index, edge_attr, W1, b1, W2, b2, root, bias):
    n, d = x.shape
    e = edge_attr.shape[0]
    e_pad = NW * EPW

    src = edge_index[0].astype(jnp.int32)
    dst = edge_index[1].astype(jnp.int32)
    src_p = jnp.pad(src, (0, e_pad - e))
    dst_p = jnp.pad(dst, (0, e_pad - e), constant_values=n)
    ea_p = jnp.pad(edge_attr, ((0, e_pad - e), (0, 0)))

    src_r = src_p.reshape(NW, EPW // ROWCH, ROWCH)
    idxr, idxhi = [], []
    for c in range(NC):
        loc = dst_p - c * HALF
        ok = (loc >= 0) & (loc < HALF)
        idxr.append(jnp.where(ok, loc, HALF))
        idxhi.append(jnp.where(ok, loc // D, HALF // D))
    idxr = jnp.stack(idxr).reshape(NC, NS, 2 * EPW // ROWCH, ROWCH)
    idxhi = jnp.stack(idxhi).reshape(NC, NS, 2 * EPW // ROWCH, ROWCH)
    dmod = (dst_p % D).reshape(e_pad, 1)
    zeros = jnp.zeros((ACC1 // NS, D), jnp.float32)

    M = W2.reshape(HID * D, D)
    B2 = b2.reshape(D, D)
    b1r = b1.reshape(1, HID)
    biasr = bias.reshape(1, D)

    xj = _sc_gather(x, src_r, e_pad)
    msg, hot = _tc_edge(ea_p, xj, dmod, W1, b1r, M, B2, e_pad)
    part, cnthot = _sc_scatter(msg, hot, idxr, idxhi, zeros)
    summed = jnp.concatenate([part[0, :HALF], part[1, :HALF]])[:n]
    cnt = jnp.concatenate(
        [cnthot[0, :HALF // D], cnthot[1, :HALF // D]]).reshape(-1)[:n]
    return _tc_final(x, summed, cnt.reshape(n, 1), root, biasr, n)


# speedup vs baseline: 2.3258x; 2.3258x over previous
import functools

import jax
import jax.numpy as jnp
from jax import lax
from jax.experimental import pallas as pl
from jax.experimental.pallas import tpu as pltpu
from jax.experimental.pallas import tpu_sc as plsc

D = 128
HID = 128
EB = 256
KC = 16
NC = 2
NS = 16
NW = NC * NS
EPW = 640
ROWCH = 128

_HIGH = lax.Precision.HIGHEST


def _gather_body(x_hbm, src_hbm, out_hbm, idx_v, rows_v, sem):
    c = lax.axis_index("c")
    s = lax.axis_index("s")
    wid = c * NS + s
    pltpu.sync_copy(src_hbm.at[wid], idx_v)
    for j in range(EPW // ROWCH):
        pltpu.async_copy(x_hbm.at[idx_v.at[j]],
                         rows_v.at[pl.ds(j * ROWCH, ROWCH)], sem).wait()
    pltpu.sync_copy(rows_v, out_hbm.at[pl.ds(wid * EPW, EPW)])


def _sc_gather(x, src_r, e_pad):
    mesh = plsc.VectorSubcoreMesh(core_axis_name="c", subcore_axis_name="s")
    k = pl.kernel(
        _gather_body,
        out_type=jax.ShapeDtypeStruct((e_pad, D), jnp.float32),
        mesh=mesh,
        scratch_types=[
            pltpu.VMEM((EPW // ROWCH, ROWCH), jnp.int32),
            pltpu.VMEM((EPW, D), jnp.float32),
            pltpu.SemaphoreType.DMA,
        ],
    )
    return k(x, src_r)


HALF = 5120
ACC1 = 5248
ACC2 = 128


def _scatter_body(msg_hbm, hot_hbm, idxr_hbm, idxhi_hbm, zero_hbm,
                  out1_hbm, out2_hbm, idx_v, hi_v, rows_v, acc_sh, cnt_sh):
    per1 = ACC1 // NS
    per2 = ACC2 // NS
    c = lax.axis_index("c")
    s = lax.axis_index("s")

    pltpu.sync_copy(zero_hbm, acc_sh.at[pl.ds(s * per1, per1)])
    pltpu.sync_copy(zero_hbm.at[pl.ds(0, per2)],
                    cnt_sh.at[pl.ds(s * per2, per2)])
    pltpu.sync_copy(idxr_hbm.at[c, s], idx_v)
    pltpu.sync_copy(idxhi_hbm.at[c, s], hi_v)
    plsc.subcore_barrier()

    for chunk in range(2):
        base = s * 2 * EPW + chunk * EPW
        pltpu.sync_copy(msg_hbm.at[pl.ds(base, EPW)], rows_v)
        for j in range(EPW // ROWCH):
            pltpu.sync_copy(rows_v.at[pl.ds(j * ROWCH, ROWCH)],
                            acc_sh.at[idx_v.at[chunk * (EPW // ROWCH) + j]],
                            add=True)
        pltpu.sync_copy(hot_hbm.at[pl.ds(base, EPW)], rows_v)
        for j in range(EPW // ROWCH):
            pltpu.sync_copy(rows_v.at[pl.ds(j * ROWCH, ROWCH)],
                            cnt_sh.at[hi_v.at[chunk * (EPW // ROWCH) + j]],
                            add=True)

    plsc.subcore_barrier()
    pltpu.sync_copy(acc_sh.at[pl.ds(s * per1, per1)],
                    out1_hbm.at[c, pl.ds(s * per1, per1)])
    pltpu.sync_copy(cnt_sh.at[pl.ds(s * per2, per2)],
                    out2_hbm.at[c, pl.ds(s * per2, per2)])


def _sc_scatter(msg, hot, idxr, idxhi, zeros):
    mesh = plsc.VectorSubcoreMesh(core_axis_name="c", subcore_axis_name="s")
    k = pl.kernel(
        _scatter_body,
        out_type=(jax.ShapeDtypeStruct((NC, ACC1, D), jnp.float32),
                  jax.ShapeDtypeStruct((NC, ACC2, D), jnp.float32)),
        mesh=mesh,
        scratch_types=[
            pltpu.VMEM((2 * EPW // ROWCH, ROWCH), jnp.int32),
            pltpu.VMEM((2 * EPW // ROWCH, ROWCH), jnp.int32),
            pltpu.VMEM((EPW, D), jnp.float32),
            pltpu.VMEM_SHARED((ACC1, D), jnp.float32),
            pltpu.VMEM_SHARED((ACC2, D), jnp.float32),
        ],
    )
    return k(msg, hot, idxr, idxhi, zeros)


def _edge_kernel(ea_ref, xj_ref, dm_ref, W1_ref, b1_ref, M_ref, B2_ref,
                 msg_ref, hot_ref, gc_ref):
    h = jnp.maximum(
        jnp.dot(ea_ref[...], W1_ref[...], preferred_element_type=jnp.float32,
                precision=_HIGH) + b1_ref[...], 0.0)
    xj = xj_ref[...]
    acc = jnp.dot(xj, B2_ref[...], preferred_element_type=jnp.float32,
                  precision=_HIGH)
    for kc in range(0, HID, KC):
        for j in range(KC):
            gc_ref[:, j * D:(j + 1) * D] = h[:, kc + j:kc + j + 1] * xj
        acc = acc + jnp.dot(
            gc_ref[...], M_ref[kc * D:(kc + KC) * D, :],
            preferred_element_type=jnp.float32)
    msg_ref[...] = acc
    lanes = lax.broadcasted_iota(jnp.int32, (EB, D), 1)
    hot_ref[...] = jnp.where(lanes == dm_ref[...], 1.0, 0.0)


def _tc_edge(ea, xj, dmod, W1, b1, M, B2, e_pad):
    grid = e_pad // EB
    return pl.pallas_call(
        _edge_kernel,
        grid=(grid,),
        in_specs=[
            pl.BlockSpec((EB, D), lambda i: (i, 0)),
            pl.BlockSpec((EB, D), lambda i: (i, 0)),
            pl.BlockSpec((EB, 1), lambda i: (i, 0)),
            pl.BlockSpec((D, HID), lambda i: (0, 0)),
            pl.BlockSpec((1, HID), lambda i: (0, 0)),
            pl.BlockSpec((HID * D, D), lambda i: (0, 0)),
            pl.BlockSpec((D, D), lambda i: (0, 0)),
        ],
        out_specs=[
            pl.BlockSpec((EB, D), lambda i: (i, 0)),
            pl.BlockSpec((EB, D), lambda i: (i, 0)),
        ],
        out_shape=[
            jax.ShapeDtypeStruct((e_pad, D), jnp.float32),
            jax.ShapeDtypeStruct((e_pad, D), jnp.float32),
        ],
        scratch_shapes=[pltpu.VMEM((EB, KC * D), jnp.float32)],
    )(ea, xj, dmod, W1, b1, M, B2)


def _final_kernel(x_ref, s_ref, cnt_ref, root_ref, bias_ref, out_ref):
    x = x_ref[...]
    aggr = s_ref[...] / jnp.maximum(cnt_ref[...], 1.0)
    pre = aggr + jnp.dot(x, root_ref[...], preferred_element_type=jnp.float32,
                         precision=_HIGH) + bias_ref[...]
    out_ref[...] = x + 0.5 * pre * (1.0 + lax.erf(pre * 0.7071067811865476))


def _tc_final(x, summed, cnt, root, bias, n):
    nb = 1000
    return pl.pallas_call(
        _final_kernel,
        grid=(n // nb,),
        in_specs=[
            pl.BlockSpec((nb, D), lambda i: (i, 0)),
            pl.BlockSpec((nb, D), lambda i: (i, 0)),
            pl.BlockSpec((nb, 1), lambda i: (i, 0)),
            pl.BlockSpec((D, D), lambda i: (0, 0)),
            pl.BlockSpec((1, D), lambda i: (0, 0)),
        ],
        out_specs=pl.BlockSpec((nb, D), lambda i: (i, 0)),
        out_shape=jax.ShapeDtypeStruct((n, D), jnp.float32),
    )(x, summed, cnt, root, bias)


def kernel(x, edge_index, edge_attr, W1, b1, W2, b2, root, bias):
    n, d = x.shape
    e = edge_attr.shape[0]
    e_pad = NW * EPW

    src = edge_index[0].astype(jnp.int32)
    dst = edge_index[1].astype(jnp.int32)
    src_p = jnp.pad(src, (0, e_pad - e))
    dst_p = jnp.pad(dst, (0, e_pad - e), constant_values=n)
    ea_p = jnp.pad(edge_attr, ((0, e_pad - e), (0, 0)))

    src_r = src_p.reshape(NW, EPW // ROWCH, ROWCH)
    idxr, idxhi = [], []
    for c in range(NC):
        loc = dst_p - c * HALF
        ok = (loc >= 0) & (loc < HALF)
        idxr.append(jnp.where(ok, loc, HALF))
        idxhi.append(jnp.where(ok, loc // D, HALF // D))
    idxr = jnp.stack(idxr).reshape(NC, NS, 2 * EPW // ROWCH, ROWCH)
    idxhi = jnp.stack(idxhi).reshape(NC, NS, 2 * EPW // ROWCH, ROWCH)
    dmod = (dst_p % D).reshape(e_pad, 1)
    zeros = jnp.zeros((ACC1 // NS, D), jnp.float32)

    M = W2.reshape(HID * D, D)
    B2 = b2.reshape(D, D)
    b1r = b1.reshape(1, HID)
    biasr = bias.reshape(1, D)

    xj = _sc_gather(x, src_r, e_pad)
    msg, hot = _tc_edge(ea_p, xj, dmod, W1, b1r, M, B2, e_pad)
    part, cnthot = _sc_scatter(msg, hot, idxr, idxhi, zeros)
    summed = jnp.concatenate([part[0, :HALF], part[1, :HALF]])[:n]
    cnt = jnp.concatenate(
        [cnthot[0, :HALF // D], cnthot[1, :HALF // D]]).reshape(-1)[:n]
    return _tc_final(x, summed, cnt.reshape(n, 1), root, biasr, n)


# speedup vs baseline: 2.7277x; 1.1728x over previous
import functools

import jax
import jax.numpy as jnp
from jax import lax
from jax.experimental import pallas as pl
from jax.experimental.pallas import tpu as pltpu
from jax.experimental.pallas import tpu_sc as plsc

D = 128
HID = 128
EB = 512
KC = 16
NC = 2
NS = 16
NW = NC * NS
EPW = 640
ROWCH = 128

_HIGH = lax.Precision.HIGHEST


def _gather_body(x_hbm, src_hbm, out_hbm, idx_v, rows_v, sem):
    c = lax.axis_index("c")
    s = lax.axis_index("s")
    wid = c * NS + s
    pltpu.sync_copy(src_hbm.at[wid], idx_v)
    for j in range(EPW // ROWCH):
        pltpu.async_copy(x_hbm.at[idx_v.at[j]],
                         rows_v.at[pl.ds(j * ROWCH, ROWCH)], sem).wait()
    pltpu.sync_copy(rows_v, out_hbm.at[pl.ds(wid * EPW, EPW)])


def _sc_gather(x, src_r, e_pad):
    mesh = plsc.VectorSubcoreMesh(core_axis_name="c", subcore_axis_name="s")
    k = pl.kernel(
        _gather_body,
        out_type=jax.ShapeDtypeStruct((e_pad, D), jnp.float32),
        mesh=mesh,
        scratch_types=[
            pltpu.VMEM((EPW // ROWCH, ROWCH), jnp.int32),
            pltpu.VMEM((EPW, D), jnp.float32),
            pltpu.SemaphoreType.DMA,
        ],
    )
    return k(x, src_r)


HALF = 5120
ACC1 = 5248
ACC2 = 128


def _scatter_body(msg_hbm, hot_hbm, idxr_hbm, idxhi_hbm, zero_hbm,
                  out1_hbm, out2_hbm, idx_v, hi_v, rows_v, acc_sh, cnt_sh):
    per1 = ACC1 // NS
    per2 = ACC2 // NS
    c = lax.axis_index("c")
    s = lax.axis_index("s")

    pltpu.sync_copy(zero_hbm, acc_sh.at[pl.ds(s * per1, per1)])
    pltpu.sync_copy(zero_hbm.at[pl.ds(0, per2)],
                    cnt_sh.at[pl.ds(s * per2, per2)])
    pltpu.sync_copy(idxr_hbm.at[c, s], idx_v)
    pltpu.sync_copy(idxhi_hbm.at[c, s], hi_v)
    plsc.subcore_barrier()

    for chunk in range(2):
        base = s * 2 * EPW + chunk * EPW
        pltpu.sync_copy(msg_hbm.at[pl.ds(base, EPW)], rows_v)
        for j in range(EPW // ROWCH):
            pltpu.sync_copy(rows_v.at[pl.ds(j * ROWCH, ROWCH)],
                            acc_sh.at[idx_v.at[chunk * (EPW // ROWCH) + j]],
                            add=True)
        pltpu.sync_copy(hot_hbm.at[pl.ds(base, EPW)], rows_v)
        for j in range(EPW // ROWCH):
            pltpu.sync_copy(rows_v.at[pl.ds(j * ROWCH, ROWCH)],
                            cnt_sh.at[hi_v.at[chunk * (EPW // ROWCH) + j]],
                            add=True)

    plsc.subcore_barrier()
    pltpu.sync_copy(acc_sh.at[pl.ds(s * per1, per1)],
                    out1_hbm.at[c, pl.ds(s * per1, per1)])
    pltpu.sync_copy(cnt_sh.at[pl.ds(s * per2, per2)],
                    out2_hbm.at[c, pl.ds(s * per2, per2)])


def _sc_scatter(msg, hot, idxr, idxhi, zeros):
    mesh = plsc.VectorSubcoreMesh(core_axis_name="c", subcore_axis_name="s")
    k = pl.kernel(
        _scatter_body,
        out_type=(jax.ShapeDtypeStruct((NC, ACC1, D), jnp.float32),
                  jax.ShapeDtypeStruct((NC, ACC2, D), jnp.float32)),
        mesh=mesh,
        scratch_types=[
            pltpu.VMEM((2 * EPW // ROWCH, ROWCH), jnp.int32),
            pltpu.VMEM((2 * EPW // ROWCH, ROWCH), jnp.int32),
            pltpu.VMEM((EPW, D), jnp.float32),
            pltpu.VMEM_SHARED((ACC1, D), jnp.float32),
            pltpu.VMEM_SHARED((ACC2, D), jnp.float32),
        ],
    )
    return k(msg, hot, idxr, idxhi, zeros)


def _edge_kernel(ea_ref, xj_ref, dm_ref, W1_ref, b1_ref, M_ref, B2_ref,
                 msg_ref, hot_ref, gc_ref):
    h = jnp.maximum(
        jnp.dot(ea_ref[...], W1_ref[...], preferred_element_type=jnp.float32,
                precision=_HIGH) + b1_ref[...], 0.0)
    xj = xj_ref[...]
    acc = jnp.dot(xj, B2_ref[...], preferred_element_type=jnp.float32,
                  precision=_HIGH)
    hb = h.astype(jnp.bfloat16)
    xb = xj.astype(jnp.bfloat16)
    for kc in range(0, HID, KC):
        for j in range(KC):
            gc_ref[:, j * D:(j + 1) * D] = hb[:, kc + j:kc + j + 1] * xb
        acc = acc + jnp.dot(
            gc_ref[...], M_ref[kc * D:(kc + KC) * D, :],
            preferred_element_type=jnp.float32)
    msg_ref[...] = acc
    lanes = lax.broadcasted_iota(jnp.int32, (EB, D), 1)
    hot_ref[...] = jnp.where(lanes == dm_ref[...], 1.0, 0.0)


def _tc_edge(ea, xj, dmod, W1, b1, M, B2, e_pad):
    grid = e_pad // EB
    return pl.pallas_call(
        _edge_kernel,
        grid=(grid,),
        in_specs=[
            pl.BlockSpec((EB, D), lambda i: (i, 0)),
            pl.BlockSpec((EB, D), lambda i: (i, 0)),
            pl.BlockSpec((EB, 1), lambda i: (i, 0)),
            pl.BlockSpec((D, HID), lambda i: (0, 0)),
            pl.BlockSpec((1, HID), lambda i: (0, 0)),
            pl.BlockSpec((HID * D, D), lambda i: (0, 0)),
            pl.BlockSpec((D, D), lambda i: (0, 0)),
        ],
        out_specs=[
            pl.BlockSpec((EB, D), lambda i: (i, 0)),
            pl.BlockSpec((EB, D), lambda i: (i, 0)),
        ],
        out_shape=[
            jax.ShapeDtypeStruct((e_pad, D), jnp.float32),
            jax.ShapeDtypeStruct((e_pad, D), jnp.float32),
        ],
        scratch_shapes=[pltpu.VMEM((EB, KC * D), jnp.bfloat16)],
    )(ea, xj, dmod, W1, b1, M, B2)


def _final_kernel(x_ref, s_ref, cnt_ref, root_ref, bias_ref, out_ref):
    x = x_ref[...]
    aggr = s_ref[...] / jnp.maximum(cnt_ref[...], 1.0)
    pre = aggr + jnp.dot(x, root_ref[...], preferred_element_type=jnp.float32,
                         precision=_HIGH) + bias_ref[...]
    out_ref[...] = x + 0.5 * pre * (1.0 + lax.erf(pre * 0.7071067811865476))


def _tc_final(x, summed, cnt, root, bias, n):
    nb = 1000
    return pl.pallas_call(
        _final_kernel,
        grid=(n // nb,),
        in_specs=[
            pl.BlockSpec((nb, D), lambda i: (i, 0)),
            pl.BlockSpec((nb, D), lambda i: (i, 0)),
            pl.BlockSpec((nb, 1), lambda i: (i, 0)),
            pl.BlockSpec((D, D), lambda i: (0, 0)),
            pl.BlockSpec((1, D), lambda i: (0, 0)),
        ],
        out_specs=pl.BlockSpec((nb, D), lambda i: (i, 0)),
        out_shape=jax.ShapeDtypeStruct((n, D), jnp.float32),
    )(x, summed, cnt, root, bias)


def kernel(x, edge_index, edge_attr, W1, b1, W2, b2, root, bias):
    n, d = x.shape
    e = edge_attr.shape[0]
    e_pad = NW * EPW

    src = edge_index[0].astype(jnp.int32)
    dst = edge_index[1].astype(jnp.int32)
    src_p = jnp.pad(src, (0, e_pad - e))
    dst_p = jnp.pad(dst, (0, e_pad - e), constant_values=n)
    ea_p = jnp.pad(edge_attr, ((0, e_pad - e), (0, 0)))

    src_r = src_p.reshape(NW, EPW // ROWCH, ROWCH)
    idxr, idxhi = [], []
    for c in range(NC):
        loc = dst_p - c * HALF
        ok = (loc >= 0) & (loc < HALF)
        idxr.append(jnp.where(ok, loc, HALF))
        idxhi.append(jnp.where(ok, loc // D, HALF // D))
    idxr = jnp.stack(idxr).reshape(NC, NS, 2 * EPW // ROWCH, ROWCH)
    idxhi = jnp.stack(idxhi).reshape(NC, NS, 2 * EPW // ROWCH, ROWCH)
    dmod = (dst_p % D).reshape(e_pad, 1)
    zeros = jnp.zeros((ACC1 // NS, D), jnp.float32)

    M = W2.reshape(HID * D, D).astype(jnp.bfloat16)
    B2 = b2.reshape(D, D)
    b1r = b1.reshape(1, HID)
    biasr = bias.reshape(1, D)

    xj = _sc_gather(x, src_r, e_pad)
    msg, hot = _tc_edge(ea_p, xj, dmod, W1, b1r, M, B2, e_pad)
    part, cnthot = _sc_scatter(msg, hot, idxr, idxhi, zeros)
    summed = jnp.concatenate([part[0, :HALF], part[1, :HALF]])[:n]
    cnt = jnp.concatenate(
        [cnthot[0, :HALF // D], cnthot[1, :HALF // D]]).reshape(-1)[:n]
    return _tc_final(x, summed, cnt.reshape(n, 1), root, biasr, n)


# speedup vs baseline: 4.5503x; 1.6682x over previous
import functools

import jax
import jax.numpy as jnp
from jax import lax
from jax.experimental import pallas as pl
from jax.experimental.pallas import tpu as pltpu
from jax.experimental.pallas import tpu_sc as plsc

D = 128
HID = 128
EB = 512
KC = 16
NC = 2
NS = 16
NW = NC * NS
EPW = 640
ROWCH = 128

_HIGH = lax.Precision.HIGHEST


def _gather_body(x_hbm, src_hbm, out_hbm, idx_v, rows_v, sem):
    c = lax.axis_index("c")
    s = lax.axis_index("s")
    wid = c * NS + s
    pltpu.sync_copy(src_hbm.at[wid], idx_v)
    for j in range(EPW // ROWCH):
        pltpu.async_copy(x_hbm.at[idx_v.at[j]],
                         rows_v.at[pl.ds(j * ROWCH, ROWCH)], sem).wait()
    pltpu.sync_copy(rows_v, out_hbm.at[pl.ds(wid * EPW, EPW)])


def _sc_gather(x, src_r, e_pad):
    mesh = plsc.VectorSubcoreMesh(core_axis_name="c", subcore_axis_name="s")
    k = pl.kernel(
        _gather_body,
        out_type=jax.ShapeDtypeStruct((e_pad, D), jnp.float32),
        mesh=mesh,
        scratch_types=[
            pltpu.VMEM((EPW // ROWCH, ROWCH), jnp.int32),
            pltpu.VMEM((EPW, D), jnp.float32),
            pltpu.SemaphoreType.DMA,
        ],
    )
    return k(x, src_r)


HALF = 5120
ACC1 = 5248
ACC2 = 128


def _scatter_body(msg_hbm, hot_hbm, idxr_hbm, idxhi_hbm, zero_hbm,
                  out1_hbm, out2_hbm, idx_v, hi_v, rows_v, acc_sh, cnt_sh):
    per1 = ACC1 // NS
    per2 = ACC2 // NS
    c = lax.axis_index("c")
    s = lax.axis_index("s")

    pltpu.sync_copy(zero_hbm, acc_sh.at[pl.ds(s * per1, per1)])
    pltpu.sync_copy(zero_hbm.at[pl.ds(0, per2)],
                    cnt_sh.at[pl.ds(s * per2, per2)])
    pltpu.sync_copy(idxr_hbm.at[c, s], idx_v)
    pltpu.sync_copy(idxhi_hbm.at[c, s], hi_v)
    plsc.subcore_barrier()

    for chunk in range(2):
        base = s * 2 * EPW + chunk * EPW
        pltpu.sync_copy(msg_hbm.at[pl.ds(base, EPW)], rows_v)
        for j in range(EPW // ROWCH):
            pltpu.sync_copy(rows_v.at[pl.ds(j * ROWCH, ROWCH)],
                            acc_sh.at[idx_v.at[chunk * (EPW // ROWCH) + j]],
                            add=True)
        pltpu.sync_copy(hot_hbm.at[pl.ds(base, EPW)], rows_v)
        for j in range(EPW // ROWCH):
            pltpu.sync_copy(rows_v.at[pl.ds(j * ROWCH, ROWCH)],
                            cnt_sh.at[hi_v.at[chunk * (EPW // ROWCH) + j]],
                            add=True)

    plsc.subcore_barrier()
    pltpu.sync_copy(acc_sh.at[pl.ds(s * per1, per1)],
                    out1_hbm.at[c, pl.ds(s * per1, per1)])
    pltpu.sync_copy(cnt_sh.at[pl.ds(s * per2, per2)],
                    out2_hbm.at[c, pl.ds(s * per2, per2)])


def _sc_scatter(msg, hot, idxr, idxhi, zeros):
    mesh = plsc.VectorSubcoreMesh(core_axis_name="c", subcore_axis_name="s")
    k = pl.kernel(
        _scatter_body,
        out_type=(jax.ShapeDtypeStruct((NC, ACC1, D), jnp.float32),
                  jax.ShapeDtypeStruct((NC, ACC2, D), jnp.float32)),
        mesh=mesh,
        scratch_types=[
            pltpu.VMEM((2 * EPW // ROWCH, ROWCH), jnp.int32),
            pltpu.VMEM((2 * EPW // ROWCH, ROWCH), jnp.int32),
            pltpu.VMEM((EPW, D), jnp.float32),
            pltpu.VMEM_SHARED((ACC1, D), jnp.float32),
            pltpu.VMEM_SHARED((ACC2, D), jnp.float32),
        ],
    )
    return k(msg, hot, idxr, idxhi, zeros)


def _edge_kernel(ea_ref, xj_ref, dm_ref, W1_ref, b1_ref, Mt_ref, B2_ref,
                 msg_ref, hot_ref, gt_ref, gt2_ref):
    h = jnp.maximum(
        jnp.dot(ea_ref[...], W1_ref[...], preferred_element_type=jnp.float32,
                precision=_HIGH) + b1_ref[...], 0.0)
    xj = xj_ref[...]
    hbt = h.astype(jnp.bfloat16).T
    xbt = xj.astype(jnp.bfloat16).T
    acc_t = jnp.zeros((D, EB), jnp.float32)
    bufs = (gt_ref, gt2_ref)
    for t, kc in enumerate(range(0, HID, KC)):
        buf = bufs[t % 2]
        for j in range(KC):
            buf[j * D:(j + 1) * D, :] = hbt[kc + j:kc + j + 1, :] * xbt
        acc_t = acc_t + jnp.dot(
            Mt_ref[:, kc * D:(kc + KC) * D], buf[...],
            preferred_element_type=jnp.float32)
    acc = acc_t.T + jnp.dot(xj, B2_ref[...],
                            preferred_element_type=jnp.float32,
                            precision=_HIGH)
    msg_ref[...] = acc
    lanes = lax.broadcasted_iota(jnp.int32, (EB, D), 1)
    hot_ref[...] = jnp.where(lanes == dm_ref[...], 1.0, 0.0)


def _tc_edge(ea, xj, dmod, W1, b1, M, B2, e_pad):
    grid = e_pad // EB
    return pl.pallas_call(
        _edge_kernel,
        grid=(grid,),
        in_specs=[
            pl.BlockSpec((EB, D), lambda i: (i, 0)),
            pl.BlockSpec((EB, D), lambda i: (i, 0)),
            pl.BlockSpec((EB, 1), lambda i: (i, 0)),
            pl.BlockSpec((D, HID), lambda i: (0, 0)),
            pl.BlockSpec((1, HID), lambda i: (0, 0)),
            pl.BlockSpec((D, HID * D), lambda i: (0, 0)),
            pl.BlockSpec((D, D), lambda i: (0, 0)),
        ],
        out_specs=[
            pl.BlockSpec((EB, D), lambda i: (i, 0)),
            pl.BlockSpec((EB, D), lambda i: (i, 0)),
        ],
        out_shape=[
            jax.ShapeDtypeStruct((e_pad, D), jnp.float32),
            jax.ShapeDtypeStruct((e_pad, D), jnp.float32),
        ],
        scratch_shapes=[pltpu.VMEM((KC * D, EB), jnp.bfloat16),
                        pltpu.VMEM((KC * D, EB), jnp.bfloat16)],
    )(ea, xj, dmod, W1, b1, M, B2)


def _final_kernel(x_ref, s_ref, cnt_ref, root_ref, bias_ref, out_ref):
    x = x_ref[...]
    aggr = s_ref[...] / jnp.maximum(cnt_ref[...], 1.0)
    pre = aggr + jnp.dot(x, root_ref[...], preferred_element_type=jnp.float32,
                         precision=_HIGH) + bias_ref[...]
    out_ref[...] = x + 0.5 * pre * (1.0 + lax.erf(pre * 0.7071067811865476))


def _tc_final(x, summed, cnt, root, bias, n):
    nb = 1000
    return pl.pallas_call(
        _final_kernel,
        grid=(n // nb,),
        in_specs=[
            pl.BlockSpec((nb, D), lambda i: (i, 0)),
            pl.BlockSpec((nb, D), lambda i: (i, 0)),
            pl.BlockSpec((nb, 1), lambda i: (i, 0)),
            pl.BlockSpec((D, D), lambda i: (0, 0)),
            pl.BlockSpec((1, D), lambda i: (0, 0)),
        ],
        out_specs=pl.BlockSpec((nb, D), lambda i: (i, 0)),
        out_shape=jax.ShapeDtypeStruct((n, D), jnp.float32),
    )(x, summed, cnt, root, bias)


def kernel(x, edge_index, edge_attr, W1, b1, W2, b2, root, bias):
    n, d = x.shape
    e = edge_attr.shape[0]
    e_pad = NW * EPW

    src = edge_index[0].astype(jnp.int32)
    dst = edge_index[1].astype(jnp.int32)
    src_p = jnp.pad(src, (0, e_pad - e))
    dst_p = jnp.pad(dst, (0, e_pad - e), constant_values=n)
    ea_p = jnp.pad(edge_attr, ((0, e_pad - e), (0, 0)))

    src_r = src_p.reshape(NW, EPW // ROWCH, ROWCH)
    idxr, idxhi = [], []
    for c in range(NC):
        loc = dst_p - c * HALF
        ok = (loc >= 0) & (loc < HALF)
        idxr.append(jnp.where(ok, loc, HALF))
        idxhi.append(jnp.where(ok, loc // D, HALF // D))
    idxr = jnp.stack(idxr).reshape(NC, NS, 2 * EPW // ROWCH, ROWCH)
    idxhi = jnp.stack(idxhi).reshape(NC, NS, 2 * EPW // ROWCH, ROWCH)
    dmod = (dst_p % D).reshape(e_pad, 1)
    zeros = jnp.zeros((ACC1 // NS, D), jnp.float32)

    M = W2.reshape(HID * D, D).astype(jnp.bfloat16).T
    B2 = b2.reshape(D, D)
    b1r = b1.reshape(1, HID)
    biasr = bias.reshape(1, D)

    xj = _sc_gather(x, src_r, e_pad)
    msg, hot = _tc_edge(ea_p, xj, dmod, W1, b1r, M, B2, e_pad)
    part, cnthot = _sc_scatter(msg, hot, idxr, idxhi, zeros)
    summed = jnp.concatenate([part[0, :HALF], part[1, :HALF]])[:n]
    cnt = jnp.concatenate(
        [cnthot[0, :HALF // D], cnthot[1, :HALF // D]]).reshape(-1)[:n]
    return _tc_final(x, summed, cnt.reshape(n, 1), root, biasr, n)


# speedup vs baseline: 4.6348x; 1.0186x over previous
import jax
import jax.numpy as jnp
from jax import lax
from jax.experimental import pallas as pl
from jax.experimental.pallas import tpu as pltpu
from jax.experimental.pallas import tpu_sc as plsc

D = 128
HID = 128
EB = 512
KC = 16
NC = 2
NS = 16
NW = NC * NS
EPW = 640
ROWCH = 128

_HIGH = lax.Precision.HIGHEST


def _gather_body(x_hbm, src_hbm, out_hbm, idx_v, rows_v, sem):
    c = lax.axis_index("c")
    s = lax.axis_index("s")
    wid = c * NS + s
    pltpu.sync_copy(src_hbm.at[wid], idx_v)
    hs = []
    for j in range(EPW // ROWCH):
        hs.append(pltpu.async_copy(x_hbm.at[idx_v.at[j]],
                                   rows_v.at[pl.ds(j * ROWCH, ROWCH)], sem))
    for h in hs:
        h.wait()
    pltpu.sync_copy(rows_v, out_hbm.at[pl.ds(wid * EPW, EPW)])


def _sc_gather(x, src_r, e_pad):
    mesh = plsc.VectorSubcoreMesh(core_axis_name="c", subcore_axis_name="s")
    k = pl.kernel(
        _gather_body,
        out_type=jax.ShapeDtypeStruct((e_pad, D), jnp.float32),
        mesh=mesh,
        scratch_types=[
            pltpu.VMEM((EPW // ROWCH, ROWCH), jnp.int32),
            pltpu.VMEM((EPW, D), jnp.float32),
            pltpu.SemaphoreType.DMA,
        ],
    )
    return k(x, src_r)


HALF = 5120
ACC1 = 5248
CNT2 = 384
CH = 1


def _scatter_body(msg_hbm, hot_hbm, idxr_hbm, idxlo_hbm, zero_hbm, z16_hbm,
                  out1_hbm, out2_hbm, idx_v, lo_v, mb0, mb1, hb0, hb1,
                  acc_sh, cnt_sh, sl0, sl1):
    per1 = ACC1 // NS
    per2 = CNT2 // NS
    c = lax.axis_index("c")
    s = lax.axis_index("s")

    pltpu.sync_copy(zero_hbm, acc_sh.at[pl.ds(s * per1, per1)])
    pltpu.sync_copy(z16_hbm, cnt_sh.at[pl.ds(s * per2, per2)])
    pltpu.sync_copy(idxr_hbm.at[c, s], idx_v)
    pltpu.sync_copy(idxlo_hbm.at[c, s], lo_v)
    plsc.subcore_barrier()

    mbufs, hbufs, sls = (mb0, mb1), (hb0, hb1), (sl0, sl1)
    ngrp = 2 * EPW // ROWCH
    nch = ngrp // CH
    base0 = s * 2 * EPW

    def load(cidx, b):
        off = base0 + cidx * CH * ROWCH
        return (pltpu.async_copy(msg_hbm.at[pl.ds(off, CH * ROWCH)],
                                 mbufs[b], sls[b]),
                pltpu.async_copy(hot_hbm.at[pl.ds(off, CH * ROWCH)],
                                 hbufs[b], sls[b]))

    lh = load(0, 0)
    for cidx in range(nch):
        b = cidx % 2
        for h in lh:
            h.wait()
        if cidx + 1 < nch:
            lh = load(cidx + 1, 1 - b)
        for g in range(CH):
            gi = cidx * CH + g
            pltpu.sync_copy(mbufs[b].at[pl.ds(g * ROWCH, ROWCH)],
                            acc_sh.at[idx_v.at[gi]], add=True)
            pltpu.sync_copy(hbufs[b].at[pl.ds(g * ROWCH, ROWCH)],
                            cnt_sh.at[lo_v.at[gi]], add=True)

    plsc.subcore_barrier()
    pltpu.sync_copy(acc_sh.at[pl.ds(s * per1, per1)],
                    out1_hbm.at[c, pl.ds(s * per1, per1)])
    pltpu.sync_copy(cnt_sh.at[pl.ds(s * per2, per2)],
                    out2_hbm.at[c, pl.ds(s * per2, per2)])


def _sc_scatter(msg, hot16, idxr, idxlo, zeros, z16):
    mesh = plsc.VectorSubcoreMesh(core_axis_name="c", subcore_axis_name="s")
    k = pl.kernel(
        _scatter_body,
        out_type=(jax.ShapeDtypeStruct((NC, ACC1, D), jnp.float32),
                  jax.ShapeDtypeStruct((NC, CNT2, 16), jnp.float32)),
        mesh=mesh,
        scratch_types=[
            pltpu.VMEM((2 * EPW // ROWCH, ROWCH), jnp.int32),
            pltpu.VMEM((2 * EPW // ROWCH, ROWCH), jnp.int32),
            pltpu.VMEM((CH * ROWCH, D), jnp.float32),
            pltpu.VMEM((CH * ROWCH, D), jnp.float32),
            pltpu.VMEM((CH * ROWCH, 16), jnp.float32),
            pltpu.VMEM((CH * ROWCH, 16), jnp.float32),
            pltpu.VMEM_SHARED((ACC1, D), jnp.float32),
            pltpu.VMEM_SHARED((CNT2, 16), jnp.float32),
            pltpu.SemaphoreType.DMA,
            pltpu.SemaphoreType.DMA,
        ],
    )
    return k(msg, hot16, idxr, idxlo, zeros, z16)


def _edge_kernel(ea_ref, xj_ref, dm_ref, W1_ref, b1_ref, Mt_ref, B2_ref,
                 msg_ref, hot_ref, gt_ref, gt2_ref):
    h = jnp.maximum(
        jnp.dot(ea_ref[...], W1_ref[...], preferred_element_type=jnp.float32,
                precision=_HIGH) + b1_ref[...], 0.0)
    xj = xj_ref[...]
    hbt = h.astype(jnp.bfloat16).T
    xbt = xj.astype(jnp.bfloat16).T
    acc_t = jnp.zeros((D, EB), jnp.float32)
    bufs = (gt_ref, gt2_ref)
    for t, kc in enumerate(range(0, HID, KC)):
        buf = bufs[t % 2]
        for j in range(KC):
            buf[j * D:(j + 1) * D, :] = hbt[kc + j:kc + j + 1, :] * xbt
        acc_t = acc_t + jnp.dot(
            Mt_ref[:, kc * D:(kc + KC) * D], buf[...],
            preferred_element_type=jnp.float32)
    acc = acc_t.T + jnp.dot(xj, B2_ref[...],
                            preferred_element_type=jnp.float32,
                            precision=_HIGH)
    msg_ref[...] = acc
    lanes = lax.broadcasted_iota(jnp.int32, (EB // 8, D), 1)
    hot_ref[...] = jnp.where(lanes % 16 == dm_ref[...], 1.0, 0.0)


def _tc_edge(ea, xj, dm16, W1, b1, Mt, B2, e_pad):
    grid = e_pad // EB
    return pl.pallas_call(
        _edge_kernel,
        grid=(grid,),
        in_specs=[
            pl.BlockSpec((EB, D), lambda i: (i, 0)),
            pl.BlockSpec((EB, D), lambda i: (i, 0)),
            pl.BlockSpec((EB // 8, D), lambda i: (i, 0)),
            pl.BlockSpec((D, HID), lambda i: (0, 0)),
            pl.BlockSpec((1, HID), lambda i: (0, 0)),
            pl.BlockSpec((D, HID * D), lambda i: (0, 0)),
            pl.BlockSpec((D, D), lambda i: (0, 0)),
        ],
        out_specs=[
            pl.BlockSpec((EB, D), lambda i: (i, 0)),
            pl.BlockSpec((EB // 8, D), lambda i: (i, 0)),
        ],
        out_shape=[
            jax.ShapeDtypeStruct((e_pad, D), jnp.float32),
            jax.ShapeDtypeStruct((e_pad // 8, D), jnp.float32),
        ],
        scratch_shapes=[pltpu.VMEM((KC * D, EB), jnp.bfloat16),
                        pltpu.VMEM((KC * D, EB), jnp.bfloat16)],
    )(ea, xj, dm16, W1, b1, Mt, B2)


def _final_kernel(x_ref, s_ref, cnt_ref, root_ref, bias_ref, out_ref):
    x = x_ref[...]
    aggr = s_ref[...] / jnp.maximum(cnt_ref[...], 1.0)
    pre = aggr + jnp.dot(x, root_ref[...], preferred_element_type=jnp.float32,
                         precision=_HIGH) + bias_ref[...]
    out_ref[...] = x + 0.5 * pre * (1.0 + lax.erf(pre * 0.7071067811865476))


def _tc_final(x, summed, cnt, root, bias, n):
    nb = 1000
    return pl.pallas_call(
        _final_kernel,
        grid=(n // nb,),
        in_specs=[
            pl.BlockSpec((nb, D), lambda i: (i, 0)),
            pl.BlockSpec((nb, D), lambda i: (i, 0)),
            pl.BlockSpec((nb, 1), lambda i: (i, 0)),
            pl.BlockSpec((D, D), lambda i: (0, 0)),
            pl.BlockSpec((1, D), lambda i: (0, 0)),
        ],
        out_specs=pl.BlockSpec((nb, D), lambda i: (i, 0)),
        out_shape=jax.ShapeDtypeStruct((n, D), jnp.float32),
    )(x, summed, cnt, root, bias)


def kernel(x, edge_index, edge_attr, W1, b1, W2, b2, root, bias):
    n, d = x.shape
    e = edge_attr.shape[0]
    e_pad = NW * EPW

    src = edge_index[0].astype(jnp.int32)
    dst = edge_index[1].astype(jnp.int32)
    src_p = jnp.pad(src, (0, e_pad - e))
    dst_p = jnp.pad(dst, (0, e_pad - e), constant_values=n)
    ea_p = jnp.pad(edge_attr, ((0, e_pad - e), (0, 0)))

    src_r = src_p.reshape(NW, EPW // ROWCH, ROWCH)
    idxr, idxlo = [], []
    for c in range(NC):
        loc = dst_p - c * HALF
        ok = (loc >= 0) & (loc < HALF)
        idxr.append(jnp.where(ok, loc, HALF))
        idxlo.append(jnp.where(ok, loc // 16, HALF // 16))
    idxr = jnp.stack(idxr).reshape(NC, NS, 2 * EPW // ROWCH, ROWCH)
    idxlo = jnp.stack(idxlo).reshape(NC, NS, 2 * EPW // ROWCH, ROWCH)
    dm16 = jnp.repeat((dst_p % 16).reshape(e_pad // 8, 8), 16, axis=1)
    zeros = jnp.zeros((ACC1 // NS, D), jnp.float32)
    z16 = jnp.zeros((CNT2 // NS, 16), jnp.float32)

    Mt = W2.reshape(HID * D, D).astype(jnp.bfloat16).T
    B2 = b2.reshape(D, D)
    b1r = b1.reshape(1, HID)
    biasr = bias.reshape(1, D)

    xj = _sc_gather(x, src_r, e_pad)
    msg, hot = _tc_edge(ea_p, xj, dm16, W1, b1r, Mt, B2, e_pad)
    hot16 = hot.reshape(e_pad, 16)
    part, cnt16 = _sc_scatter(msg, hot16, idxr, idxlo, zeros, z16)
    summed = jnp.concatenate([part[0, :HALF], part[1, :HALF]])[:n]
    cnt = jnp.concatenate(
        [cnt16[0, :HALF // 16], cnt16[1, :HALF // 16]]).reshape(-1)[:n]
    return _tc_final(x, summed, cnt.reshape(n, 1), root, biasr, n)


# speedup vs baseline: 4.6901x; 1.0119x over previous
import jax
import jax.numpy as jnp
from jax import lax
from jax.experimental import pallas as pl
from jax.experimental.pallas import tpu as pltpu
from jax.experimental.pallas import tpu_sc as plsc

D = 128
HID = 128
EB = 512
KC = 16
NC = 2
NS = 16
NW = NC * NS
EPW = 640
ROWCH = 128

_HIGH = lax.Precision.HIGHEST


def _gather_body(x_hbm, src_hbm, out_hbm, idx_v, rows_v, sem):
    c = lax.axis_index("c")
    s = lax.axis_index("s")
    wid = c * NS + s
    pltpu.sync_copy(src_hbm.at[wid], idx_v)
    hs = []
    for j in range(EPW // ROWCH):
        hs.append(pltpu.async_copy(x_hbm.at[idx_v.at[j]],
                                   rows_v.at[pl.ds(j * ROWCH, ROWCH)], sem))
    for h in hs:
        h.wait()
    pltpu.sync_copy(rows_v, out_hbm.at[pl.ds(wid * EPW, EPW)])


def _sc_gather(x, src_r, e_pad):
    mesh = plsc.VectorSubcoreMesh(core_axis_name="c", subcore_axis_name="s")
    k = pl.kernel(
        _gather_body,
        out_type=jax.ShapeDtypeStruct((e_pad, D), jnp.float32),
        mesh=mesh,
        scratch_types=[
            pltpu.VMEM((EPW // ROWCH, ROWCH), jnp.int32),
            pltpu.VMEM((EPW, D), jnp.float32),
            pltpu.SemaphoreType.DMA,
        ],
    )
    return k(x, src_r)


HALF = 5120
ACC1 = 5248
CNT2 = 128
CH = 1


def _scatter_body(msg_hbm, hot_hbm, idxr_hbm, idxlo_hbm, zero_hbm, z16_hbm,
                  out1_hbm, out2_hbm, idx_v, lo_v, mb0, mb1, hb0, hb1,
                  acc_sh, cnt_sh, sl0, sl1):
    per1 = ACC1 // NS
    per2 = CNT2 // NS
    c = lax.axis_index("c")
    s = lax.axis_index("s")

    pltpu.sync_copy(zero_hbm, acc_sh.at[pl.ds(s * per1, per1)])
    pltpu.sync_copy(z16_hbm, cnt_sh.at[pl.ds(s * per2, per2)])
    pltpu.sync_copy(idxr_hbm.at[c, s], idx_v)
    pltpu.sync_copy(idxlo_hbm.at[c, s], lo_v)
    plsc.subcore_barrier()

    mbufs, hbufs, sls = (mb0, mb1), (hb0, hb1), (sl0, sl1)
    ngrp = 2 * EPW // ROWCH
    nch = ngrp // CH
    base0 = s * 2 * EPW

    def load(cidx, b):
        off = base0 + cidx * CH * ROWCH
        return (pltpu.async_copy(msg_hbm.at[pl.ds(off, CH * ROWCH)],
                                 mbufs[b], sls[b]),
                pltpu.async_copy(hot_hbm.at[pl.ds(off, CH * ROWCH)],
                                 hbufs[b], sls[b]))

    lh = load(0, 0)
    for cidx in range(nch):
        b = cidx % 2
        for h in lh:
            h.wait()
        if cidx + 1 < nch:
            lh = load(cidx + 1, 1 - b)
        for g in range(CH):
            gi = cidx * CH + g
            pltpu.sync_copy(mbufs[b].at[pl.ds(g * ROWCH, ROWCH)],
                            acc_sh.at[idx_v.at[gi]], add=True)
            pltpu.sync_copy(hbufs[b].at[pl.ds(g * ROWCH, ROWCH)],
                            cnt_sh.at[lo_v.at[gi]], add=True)

    plsc.subcore_barrier()
    pltpu.sync_copy(acc_sh.at[pl.ds(s * per1, per1)],
                    out1_hbm.at[c, pl.ds(s * per1, per1)])
    pltpu.sync_copy(cnt_sh.at[pl.ds(s * per2, per2)],
                    out2_hbm.at[c, pl.ds(s * per2, per2)])


def _sc_scatter(msg, hot16, idxr, idxlo, zeros, z16):
    mesh = plsc.VectorSubcoreMesh(core_axis_name="c", subcore_axis_name="s")
    k = pl.kernel(
        _scatter_body,
        out_type=(jax.ShapeDtypeStruct((NC, ACC1, D), jnp.float32),
                  jax.ShapeDtypeStruct((NC, CNT2, D), jnp.float32)),
        mesh=mesh,
        scratch_types=[
            pltpu.VMEM((2 * EPW // ROWCH, ROWCH), jnp.int32),
            pltpu.VMEM((2 * EPW // ROWCH, ROWCH), jnp.int32),
            pltpu.VMEM((CH * ROWCH, D), jnp.float32),
            pltpu.VMEM((CH * ROWCH, D), jnp.float32),
            pltpu.VMEM((CH * ROWCH, D), jnp.float32),
            pltpu.VMEM((CH * ROWCH, D), jnp.float32),
            pltpu.VMEM_SHARED((ACC1, D), jnp.float32),
            pltpu.VMEM_SHARED((CNT2, D), jnp.float32),
            pltpu.SemaphoreType.DMA,
            pltpu.SemaphoreType.DMA,
        ],
    )
    return k(msg, hot16, idxr, idxlo, zeros, z16)


def _edge_kernel(ea_ref, xj_ref, dm_ref, W1_ref, b1_ref, Mt_ref, B2_ref,
                 msg_ref, hot_ref, gt_ref, gt2_ref):
    h = jnp.maximum(
        jnp.dot(ea_ref[...], W1_ref[...], preferred_element_type=jnp.float32,
                precision=_HIGH) + b1_ref[...], 0.0)
    xj = xj_ref[...]
    hbt = h.astype(jnp.bfloat16).T
    xbt = xj.astype(jnp.bfloat16).T
    acc_t = jnp.zeros((D, EB), jnp.float32)
    bufs = (gt_ref, gt2_ref)
    for t, kc in enumerate(range(0, HID, KC)):
        buf = bufs[t % 2]
        for j in range(KC):
            buf[j * D:(j + 1) * D, :] = hbt[kc + j:kc + j + 1, :] * xbt
        acc_t = acc_t + jnp.dot(
            Mt_ref[:, kc * D:(kc + KC) * D], buf[...],
            preferred_element_type=jnp.float32)
    acc = acc_t.T + jnp.dot(xj, B2_ref[...],
                            preferred_element_type=jnp.float32,
                            precision=_HIGH)
    msg_ref[...] = acc
    lanes = lax.broadcasted_iota(jnp.int32, (EB, D), 1)
    hot_ref[...] = jnp.where(lanes == dm_ref[...], 1.0, 0.0)


def _tc_edge(ea, xj, dm16, W1, b1, Mt, B2, e_pad):
    grid = e_pad // EB
    return pl.pallas_call(
        _edge_kernel,
        grid=(grid,),
        in_specs=[
            pl.BlockSpec((EB, D), lambda i: (i, 0)),
            pl.BlockSpec((EB, D), lambda i: (i, 0)),
            pl.BlockSpec((EB, 1), lambda i: (i, 0)),
            pl.BlockSpec((D, HID), lambda i: (0, 0)),
            pl.BlockSpec((1, HID), lambda i: (0, 0)),
            pl.BlockSpec((D, HID * D), lambda i: (0, 0)),
            pl.BlockSpec((D, D), lambda i: (0, 0)),
        ],
        out_specs=[
            pl.BlockSpec((EB, D), lambda i: (i, 0)),
            pl.BlockSpec((EB, D), lambda i: (i, 0)),
        ],
        out_shape=[
            jax.ShapeDtypeStruct((e_pad, D), jnp.float32),
            jax.ShapeDtypeStruct((e_pad, D), jnp.float32),
        ],
        scratch_shapes=[pltpu.VMEM((KC * D, EB), jnp.bfloat16),
                        pltpu.VMEM((KC * D, EB), jnp.bfloat16)],
    )(ea, xj, dm16, W1, b1, Mt, B2)


def _final_kernel(x_ref, s_ref, cnt_ref, root_ref, bias_ref, out_ref):
    x = x_ref[...]
    aggr = s_ref[...] / jnp.maximum(cnt_ref[...], 1.0)
    pre = aggr + jnp.dot(x, root_ref[...], preferred_element_type=jnp.float32,
                         precision=_HIGH) + bias_ref[...]
    out_ref[...] = x + 0.5 * pre * (1.0 + lax.erf(pre * 0.7071067811865476))


def _tc_final(x, summed, cnt, root, bias, n):
    nb = 1000
    return pl.pallas_call(
        _final_kernel,
        grid=(n // nb,),
        in_specs=[
            pl.BlockSpec((nb, D), lambda i: (i, 0)),
            pl.BlockSpec((nb, D), lambda i: (i, 0)),
            pl.BlockSpec((nb, 1), lambda i: (i, 0)),
            pl.BlockSpec((D, D), lambda i: (0, 0)),
            pl.BlockSpec((1, D), lambda i: (0, 0)),
        ],
        out_specs=pl.BlockSpec((nb, D), lambda i: (i, 0)),
        out_shape=jax.ShapeDtypeStruct((n, D), jnp.float32),
    )(x, summed, cnt, root, bias)


def kernel(x, edge_index, edge_attr, W1, b1, W2, b2, root, bias):
    n, d = x.shape
    e = edge_attr.shape[0]
    e_pad = NW * EPW

    src = edge_index[0].astype(jnp.int32)
    dst = edge_index[1].astype(jnp.int32)
    src_p = jnp.pad(src, (0, e_pad - e))
    dst_p = jnp.pad(dst, (0, e_pad - e), constant_values=n)
    ea_p = jnp.pad(edge_attr, ((0, e_pad - e), (0, 0)))

    src_r = src_p.reshape(NW, EPW // ROWCH, ROWCH)
    idxr, idxlo = [], []
    for c in range(NC):
        loc = dst_p - c * HALF
        ok = (loc >= 0) & (loc < HALF)
        idxr.append(jnp.where(ok, loc, HALF))
        idxlo.append(jnp.where(ok, loc // D, HALF // D))
    idxr = jnp.stack(idxr).reshape(NC, NS, 2 * EPW // ROWCH, ROWCH)
    idxlo = jnp.stack(idxlo).reshape(NC, NS, 2 * EPW // ROWCH, ROWCH)
    dmod = (dst_p % D).reshape(e_pad, 1)
    zeros = jnp.zeros((ACC1 // NS, D), jnp.float32)
    z16 = jnp.zeros((CNT2 // NS, D), jnp.float32)

    Mt = W2.reshape(HID * D, D).astype(jnp.bfloat16).T
    B2 = b2.reshape(D, D)
    b1r = b1.reshape(1, HID)
    biasr = bias.reshape(1, D)

    xj = _sc_gather(x, src_r, e_pad)
    msg, hot = _tc_edge(ea_p, xj, dmod, W1, b1r, Mt, B2, e_pad)
    part, cnthot = _sc_scatter(msg, hot, idxr, idxlo, zeros, z16)
    summed = jnp.concatenate([part[0, :HALF], part[1, :HALF]])[:n]
    cnt = jnp.concatenate(
        [cnthot[0, :HALF // D], cnthot[1, :HALF // D]]).reshape(-1)[:n]
    return _tc_final(x, summed, cnt.reshape(n, 1), root, biasr, n)
